# baseline (speedup 1.0000x reference)
import jax
import jax.numpy as jnp
from jax import lax
from jax.experimental import pallas as pl
from jax.experimental.pallas import tpu as pltpu
from jax.experimental.pallas import tpu_sc as plsc

NUM_ROWS = 100000
DIM = 64
BATCH = 16384 * 50

NC = 2
NS = 16
NW = NC * NS
B_PER_W = BATCH // NW
CHUNK = 400
NBUF = 4
N_CHUNKS = B_PER_W // CHUNK
N_GROUPS = N_CHUNKS // NBUF


def _emb_body(w_hbm, idx_hbm, out_hbm, idx_v, rows_v, *sems):
    gsems, ssems = sems[:NBUF], sems[NBUF:]
    wid = lax.axis_index("s") * NC + lax.axis_index("c")
    base = wid * B_PER_W
    pltpu.sync_copy(idx_hbm.at[pl.ds(base, B_PER_W)], idx_v)

    def start_gather(c, b):
        pltpu.async_copy(
            w_hbm.at[idx_v.at[pl.ds(c * CHUNK, CHUNK)]], rows_v.at[b],
            gsems[b])

    def wait_gather(c, b):
        pltpu.make_async_copy(
            w_hbm.at[idx_v.at[pl.ds(c * CHUNK, CHUNK)]], rows_v.at[b],
            gsems[b]).wait()

    def start_store(c, b):
        pltpu.async_copy(
            rows_v.at[b], out_hbm.at[pl.ds(base + c * CHUNK, CHUNK)],
            ssems[b])

    def wait_store(c, b):
        pltpu.make_async_copy(
            rows_v.at[b], out_hbm.at[pl.ds(base + c * CHUNK, CHUNK)],
            ssems[b]).wait()

    for b in range(NBUF):
        start_gather(b, b)

    def group(g, _):
        c0 = g * NBUF
        for b in range(NBUF):
            wait_gather(c0 + b, b)
            start_store(c0 + b, b)
        for b in range(NBUF):
            wait_store(c0 + b, b)
            start_gather(c0 + NBUF + b, b)
        return ()

    lax.fori_loop(0, N_GROUPS - 1, group, (), unroll=False)

    c0 = (N_GROUPS - 1) * NBUF
    for b in range(NBUF):
        wait_gather(c0 + b, b)
        start_store(c0 + b, b)
    for b in range(NBUF):
        wait_store(c0 + b, b)


@jax.jit
def _embedding_sc(x_flat, W):
    mesh = plsc.VectorSubcoreMesh(core_axis_name="c", subcore_axis_name="s")
    run = pl.kernel(
        _emb_body,
        out_type=jax.ShapeDtypeStruct((BATCH, DIM), jnp.float32),
        mesh=mesh,
        scratch_types=(
            [pltpu.VMEM((B_PER_W,), jnp.int32),
             pltpu.VMEM((NBUF, CHUNK, DIM), jnp.float32)]
            + [pltpu.SemaphoreType.DMA] * (2 * NBUF)
        ),
        compiler_params=pltpu.CompilerParams(use_tc_tiling_on_sc=False),
    )
    return run(W, x_flat)


def kernel(x, W):
    x_flat = x.reshape(-1).astype(jnp.int32)
    out = _embedding_sc(x_flat, W)
    return out.reshape(x.shape + (DIM,))


# speedup vs baseline: 1.0008x; 1.0008x over previous
import jax
import jax.numpy as jnp
from jax import lax
from jax.experimental import pallas as pl
from jax.experimental.pallas import tpu as pltpu
from jax.experimental.pallas import tpu_sc as plsc

NUM_ROWS = 100000
DIM = 64
SEQS = 16384
SLEN = 50

NC = 2
NS = 16
NW = NC * NS
S_PER_W = SEQS // NW
SCHUNK = 8
RCHUNK = SCHUNK * SLEN
NBUF = 4
N_CHUNKS = S_PER_W // SCHUNK
N_GROUPS = N_CHUNKS // NBUF


def _emb_body(w_hbm, x_hbm, out_hbm, idx_v, rows_v, *sems):
    gsems, ssems = sems[:NBUF], sems[NBUF:]
    wid = lax.axis_index("s") * NC + lax.axis_index("c")
    sbase = wid * S_PER_W
    pltpu.sync_copy(x_hbm.at[wid], idx_v)

    def start_gather(c, b):
        pltpu.async_copy(
            w_hbm.at[idx_v.at[c]], rows_v.at[b], gsems[b])

    def wait_gather(c, b):
        pltpu.make_async_copy(
            w_hbm.at[idx_v.at[c]], rows_v.at[b], gsems[b]).wait()

    def start_store(c, b):
        for k in range(SCHUNK):
            pltpu.async_copy(
                rows_v.at[b, pl.ds(k * SLEN, SLEN), :],
                out_hbm.at[sbase + c * SCHUNK + k],
                ssems[b])

    def wait_store(c, b):
        for k in range(SCHUNK):
            pltpu.make_async_copy(
                rows_v.at[b, pl.ds(k * SLEN, SLEN), :],
                out_hbm.at[sbase + c * SCHUNK + k],
                ssems[b]).wait()

    for b in range(NBUF):
        start_gather(b, b)

    def group(g, _):
        c0 = g * NBUF
        for b in range(NBUF):
            wait_gather(c0 + b, b)
            start_store(c0 + b, b)
        for b in range(NBUF):
            wait_store(c0 + b, b)
            start_gather(c0 + NBUF + b, b)
        return ()

    lax.fori_loop(0, N_GROUPS - 1, group, (), unroll=False)

    c0 = (N_GROUPS - 1) * NBUF
    for b in range(NBUF):
        wait_gather(c0 + b, b)
        start_store(c0 + b, b)
    for b in range(NBUF):
        wait_store(c0 + b, b)


@jax.jit
def _embedding_sc(x2, W):
    mesh = plsc.VectorSubcoreMesh(core_axis_name="c", subcore_axis_name="s")
    run = pl.kernel(
        _emb_body,
        out_type=jax.ShapeDtypeStruct((SEQS, SLEN, DIM), jnp.float32),
        mesh=mesh,
        scratch_types=(
            [pltpu.VMEM((N_CHUNKS, RCHUNK), jnp.int32),
             pltpu.VMEM((NBUF, RCHUNK, DIM), jnp.float32)]
            + [pltpu.SemaphoreType.DMA] * (2 * NBUF)
        ),
        compiler_params=pltpu.CompilerParams(use_tc_tiling_on_sc=False),
    )
    return run(W, x2)


def kernel(x, W):
    x2 = x.astype(jnp.int32).reshape(NW, N_CHUNKS, RCHUNK)
    return _embedding_sc(x2, W)


# speedup vs baseline: 1.2532x; 1.2522x over previous
import jax
import jax.numpy as jnp
from jax import lax
from jax.experimental import pallas as pl
from jax.experimental.pallas import tpu as pltpu
from jax.experimental.pallas import tpu_sc as plsc

NUM_ROWS = 100000
DIM = 64
SEQS = 16384
SLEN = 50

NC = 2
NS = 16
NW = NC * NS
IBLK = 128
BLK_PER_W = SEQS // (NW * IBLK)
S_PER_W = IBLK * BLK_PER_W
N_CELLS = SLEN * BLK_PER_W
NBUF = 4
N_GROUPS = N_CELLS // NBUF

TBLK = 2048
TB2 = TBLK // 2


def _emb_body(w_hbm, xt_hbm, g_hbm, idx_v, g_v, *sems):
    gsems, ssems = sems[:NBUF], sems[NBUF:]
    wid = lax.axis_index("s") * NC + lax.axis_index("c")
    sbase = wid * S_PER_W
    pltpu.sync_copy(xt_hbm.at[:, pl.ds(sbase, S_PER_W)], idx_v)

    def cell(c):
        j = c // BLK_PER_W
        itl = c % BLK_PER_W
        i0 = sbase + itl * IBLK
        p0 = (i0 // TBLK) * TB2 + (i0 % TBLK) % TB2
        half = (i0 % TBLK) // TB2
        return j, itl, p0, half

    def start_gather(c, b):
        j, itl, _, _ = cell(c)
        pltpu.async_copy(
            w_hbm.at[idx_v.at[j, pl.ds(itl * IBLK, IBLK)]], g_v.at[b],
            gsems[b])

    def wait_gather(c, b):
        j, itl, _, _ = cell(c)
        pltpu.make_async_copy(
            w_hbm.at[idx_v.at[j, pl.ds(itl * IBLK, IBLK)]], g_v.at[b],
            gsems[b]).wait()

    def start_store(c, b):
        j, _, p0, half = cell(c)
        pltpu.async_copy(
            g_v.at[b],
            g_hbm.at[j, pl.ds(p0, IBLK), pl.ds(half * DIM, DIM)],
            ssems[b])

    def wait_store(c, b):
        j, _, p0, half = cell(c)
        pltpu.make_async_copy(
            g_v.at[b],
            g_hbm.at[j, pl.ds(p0, IBLK), pl.ds(half * DIM, DIM)],
            ssems[b]).wait()

    for b in range(NBUF):
        start_gather(b, b)

    def group(g, _):
        c0 = g * NBUF
        for b in range(NBUF):
            wait_gather(c0 + b, b)
            start_store(c0 + b, b)
        for b in range(NBUF):
            wait_store(c0 + b, b)
            start_gather(c0 + NBUF + b, b)
        return ()

    lax.fori_loop(0, N_GROUPS - 1, group, (), unroll=False)

    c0 = (N_GROUPS - 1) * NBUF
    for b in range(NBUF):
        wait_gather(c0 + b, b)
        start_store(c0 + b, b)
    for b in range(NBUF):
        wait_store(c0 + b, b)


def _tr_body(g_ref, o_ref):
    v = g_ref[0]
    a = v[:, 0:DIM].T
    b = v[:, DIM:2 * DIM].T
    o_ref[0] = jnp.concatenate([a, b], axis=1)


@jax.jit
def _embedding_sc(xt, W):
    mesh = plsc.VectorSubcoreMesh(core_axis_name="c", subcore_axis_name="s")
    gather = pl.kernel(
        _emb_body,
        out_type=jax.ShapeDtypeStruct((SLEN, SEQS * DIM // 128, 128),
                                      jnp.float32),
        mesh=mesh,
        scratch_types=(
            [pltpu.VMEM((SLEN, S_PER_W), jnp.int32),
             pltpu.VMEM((NBUF, IBLK, DIM), jnp.float32)]
            + [pltpu.SemaphoreType.DMA] * (2 * NBUF)
        ),
        compiler_params=pltpu.CompilerParams(use_tc_tiling_on_sc=False),
    )
    g2 = gather(W, xt)
    out_t = pl.pallas_call(
        _tr_body,
        out_shape=jax.ShapeDtypeStruct((SLEN, DIM, SEQS), jnp.float32),
        grid=(SLEN, SEQS // TBLK),
        in_specs=[pl.BlockSpec((1, TB2, 128), lambda j, i: (j, i, 0))],
        out_specs=pl.BlockSpec((1, DIM, TBLK), lambda j, i: (j, 0, i)),
    )(g2)
    return out_t


def kernel(x, W):
    xt = jnp.transpose(x.astype(jnp.int32), (1, 0))
    out_t = _embedding_sc(xt, W)
    return jnp.transpose(out_t, (2, 0, 1))
